# baseline (speedup 1.0000x reference)
import functools

import jax
import jax.numpy as jnp
from jax import lax
from jax.experimental import pallas as pl
from jax.experimental.pallas import tpu as pltpu
from jax.experimental.pallas import tpu_sc as plsc

K_ROWS = 8192
CODE_DIM = 256
EMBED_DIM = 512
OUT_DIM = 512
N_TOKENS = 8192

NCHUNK = 1
CHUNK = N_TOKENS // NCHUNK
IDS_ROWS = 64

NC, NS = 2, 16
NW = NC * NS
B_PER_W = CHUNK // NW
IDX_CHUNK = 128


NJ = B_PER_W // IDX_CHUNK


ROWS_PER_W = (IDS_ROWS // NCHUNK) // NW


def _gather_body(table_hbm, idx_hbm, out_hbm, idx2_v, idx_v, rows_v, *sems):
    gsems = sems[:NJ]
    ssems = sems[NJ:]
    wid = lax.axis_index("s") * NC + lax.axis_index("c")
    base = wid * B_PER_W
    pltpu.sync_copy(idx_hbm.at[pl.ds(wid * ROWS_PER_W, ROWS_PER_W)], idx2_v)
    ncol = idx2_v.shape[1]
    for r in range(ROWS_PER_W):
        for c0 in range(0, ncol, 16):
            idx_v[pl.ds(r * ncol + c0, 16)] = idx2_v[r, pl.ds(c0, 16)]
    gathers = []
    for j in range(NJ):
        gathers.append(
            pltpu.async_copy(
                table_hbm.at[idx_v.at[pl.ds(j * IDX_CHUNK, IDX_CHUNK)]],
                rows_v.at[pl.ds(j * IDX_CHUNK, IDX_CHUNK)],
                gsems[j],
            )
        )
    stores = []
    for j in range(NJ):
        gathers[j].wait()
        stores.append(
            pltpu.async_copy(
                rows_v.at[pl.ds(j * IDX_CHUNK, IDX_CHUNK)],
                out_hbm.at[pl.ds(base + j * IDX_CHUNK, IDX_CHUNK)],
                ssems[j],
            )
        )
    for s in stores:
        s.wait()


@functools.cache
def _sc_gather_fn():
    return pl.kernel(
        _gather_body,
        out_type=jax.ShapeDtypeStruct((CHUNK, CODE_DIM), jnp.float32),
        mesh=plsc.VectorSubcoreMesh(core_axis_name="c", subcore_axis_name="s"),
        scratch_types=[
            pltpu.VMEM((ROWS_PER_W, 128), jnp.int32),
            pltpu.VMEM((B_PER_W,), jnp.int32),
            pltpu.VMEM((B_PER_W, CODE_DIM), jnp.float32),
        ] + [pltpu.SemaphoreType.DMA] * (2 * NJ),
    )


def _flat_body(in_ref, out_ref):
    x = in_ref[...].reshape(64, 2, 64)
    e0 = x[:, 0, :]
    e1 = x[:, 1, :]
    out_ref[...] = jnp.concatenate([e0, e1], axis=1)


def _flatten_ids(ids):
    return pl.pallas_call(
        _flat_body,
        out_shape=jax.ShapeDtypeStruct((IDS_ROWS, 128), jnp.int32),
    )(ids)


def _fuse_body(wp_ref, wo_ref, wd_ref, bp_ref, bo_ref, bd_ref,
               wf_ref, bf_ref):
    t = jnp.dot(wp_ref[...], wo_ref[...], preferred_element_type=jnp.float32)
    wf_ref[...] = jnp.dot(t, wd_ref[...], preferred_element_type=jnp.float32)
    tb = jnp.dot(bp_ref[...], wo_ref[...],
                 preferred_element_type=jnp.float32) + bo_ref[...]
    bf_ref[...] = jnp.dot(tb, wd_ref[...],
                          preferred_element_type=jnp.float32) + bd_ref[...]


def _fuse_weights(W_proj, b_proj, W_out, b_out, W_dec, b_dec):
    return pl.pallas_call(
        _fuse_body,
        out_shape=(
            jax.ShapeDtypeStruct((CODE_DIM, OUT_DIM), jnp.float32),
            jax.ShapeDtypeStruct((1, OUT_DIM), jnp.float32),
        ),
    )(W_proj, W_out, W_dec,
      b_proj.reshape(1, EMBED_DIM), b_out.reshape(1, EMBED_DIM),
      b_dec.reshape(1, OUT_DIM))


M_BLK = 4096


def _mm_body(emb_ref, wf_ref, bf_ref, *rest):
    out_ref = rest[-1]
    out_ref[...] = jnp.dot(emb_ref[...], wf_ref[...],
                           preferred_element_type=jnp.float32) + bf_ref[...]


def _tc_decode_chunk(c, emb, wf, bf, prev=None):
    blk0 = c * (CHUNK // M_BLK)
    in_specs = [
        pl.BlockSpec((M_BLK, CODE_DIM), lambda i: (i, 0)),
        pl.BlockSpec((CODE_DIM, OUT_DIM), lambda i: (0, 0)),
        pl.BlockSpec((1, OUT_DIM), lambda i: (0, 0)),
    ]
    args = [emb, wf, bf]
    aliases = {}
    if prev is not None:
        in_specs.append(pl.BlockSpec(memory_space=pl.ANY))
        args.append(prev)
        aliases = {3: 0}
    return pl.pallas_call(
        _mm_body,
        grid=(CHUNK // M_BLK,),
        in_specs=in_specs,
        out_specs=pl.BlockSpec((M_BLK, OUT_DIM),
                               lambda i, blk0=blk0: (i + blk0, 0)),
        out_shape=jax.ShapeDtypeStruct((N_TOKENS, OUT_DIM), jnp.float32),
        input_output_aliases=aliases,
    )(*args)


@jax.jit
def kernel(ids, codebook, W_proj, b_proj, W_out, b_out, W_dec, b_dec):
    B, T = ids.shape
    gather = _sc_gather_fn()
    ids_v = _flatten_ids(ids)
    rpc = IDS_ROWS // NCHUNK
    if NCHUNK == 1:
        embs = [gather(codebook, ids_v)]
    else:
        embs = [gather(codebook,
                       lax.slice(ids_v, (c * rpc, 0), ((c + 1) * rpc, 128)))
                for c in range(NCHUNK)]
    wf, bf = _fuse_weights(W_proj, b_proj, W_out, b_out, W_dec, b_dec)
    out = _tc_decode_chunk(0, embs[0], wf, bf)
    for c in range(1, NCHUNK):
        out = _tc_decode_chunk(c, embs[c], wf, bf, out)
    return out.reshape(B, T, OUT_DIM)


# speedup vs baseline: 1.0157x; 1.0157x over previous
import functools

import jax
import jax.numpy as jnp
from jax import lax
from jax.experimental import pallas as pl
from jax.experimental.pallas import tpu as pltpu
from jax.experimental.pallas import tpu_sc as plsc

K_ROWS = 8192
CODE_DIM = 256
EMBED_DIM = 512
OUT_DIM = 512
N_TOKENS = 8192

NCHUNK = 1
CHUNK = N_TOKENS // NCHUNK
IDS_ROWS = 64

NC, NS = 2, 16
NW = NC * NS
B_PER_W = CHUNK // NW
IDX_CHUNK = 128


NJ = B_PER_W // IDX_CHUNK


ROWS_PER_W = (IDS_ROWS // NCHUNK) // NW


def _gather_body(table_hbm, idx_hbm, out_hbm, idx2_v, idx_v, rows_v, *sems):
    gsems = sems[:NJ]
    ssems = sems[NJ:]
    wid = lax.axis_index("s") * NC + lax.axis_index("c")
    base = wid * B_PER_W
    pltpu.sync_copy(idx_hbm.at[pl.ds(wid * ROWS_PER_W, ROWS_PER_W)], idx2_v)
    ncol = idx2_v.shape[1]
    for r in range(ROWS_PER_W):
        for c0 in range(0, ncol, 16):
            idx_v[pl.ds(r * ncol + c0, 16)] = idx2_v[r, pl.ds(c0, 16)]
    gathers = []
    for j in range(NJ):
        gathers.append(
            pltpu.async_copy(
                table_hbm.at[idx_v.at[pl.ds(j * IDX_CHUNK, IDX_CHUNK)]],
                rows_v.at[pl.ds(j * IDX_CHUNK, IDX_CHUNK)],
                gsems[j],
            )
        )
    stores = []
    for j in range(NJ):
        gathers[j].wait()
        stores.append(
            pltpu.async_copy(
                rows_v.at[pl.ds(j * IDX_CHUNK, IDX_CHUNK)],
                out_hbm.at[pl.ds(base + j * IDX_CHUNK, IDX_CHUNK)],
                ssems[j],
            )
        )
    for s in stores:
        s.wait()


@functools.cache
def _sc_gather_fn():
    return pl.kernel(
        _gather_body,
        out_type=jax.ShapeDtypeStruct((CHUNK, CODE_DIM), jnp.float32),
        mesh=plsc.VectorSubcoreMesh(core_axis_name="c", subcore_axis_name="s"),
        scratch_types=[
            pltpu.VMEM((ROWS_PER_W, 128), jnp.int32),
            pltpu.VMEM((B_PER_W,), jnp.int32),
            pltpu.VMEM((B_PER_W, CODE_DIM), jnp.float32),
        ] + [pltpu.SemaphoreType.DMA] * (2 * NJ),
    )


def _fuse_body(wp_ref, wo_ref, wd_ref, bp_ref, bo_ref, bd_ref,
               wf_ref, bf_ref):
    t = jnp.dot(wp_ref[...], wo_ref[...], preferred_element_type=jnp.float32)
    wf_ref[...] = jnp.dot(t, wd_ref[...], preferred_element_type=jnp.float32)
    tb = jnp.dot(bp_ref[...], wo_ref[...],
                 preferred_element_type=jnp.float32) + bo_ref[...]
    bf_ref[...] = jnp.dot(tb, wd_ref[...],
                          preferred_element_type=jnp.float32) + bd_ref[...]


def _fuse_weights(W_proj, b_proj, W_out, b_out, W_dec, b_dec):
    return pl.pallas_call(
        _fuse_body,
        out_shape=(
            jax.ShapeDtypeStruct((CODE_DIM, OUT_DIM), jnp.float32),
            jax.ShapeDtypeStruct((1, OUT_DIM), jnp.float32),
        ),
    )(W_proj, W_out, W_dec,
      b_proj.reshape(1, EMBED_DIM), b_out.reshape(1, EMBED_DIM),
      b_dec.reshape(1, OUT_DIM))


M_BLK = 4096


def _mm_body(emb_ref, wf_ref, bf_ref, *rest):
    out_ref = rest[-1]
    out_ref[...] = jnp.dot(emb_ref[...], wf_ref[...],
                           preferred_element_type=jnp.float32) + bf_ref[...]


def _tc_decode_chunk(c, emb, wf, bf, prev=None):
    blk0 = c * (CHUNK // M_BLK)
    in_specs = [
        pl.BlockSpec((M_BLK, CODE_DIM), lambda i: (i, 0)),
        pl.BlockSpec((CODE_DIM, OUT_DIM), lambda i: (0, 0)),
        pl.BlockSpec((1, OUT_DIM), lambda i: (0, 0)),
    ]
    args = [emb, wf, bf]
    aliases = {}
    if prev is not None:
        in_specs.append(pl.BlockSpec(memory_space=pl.ANY))
        args.append(prev)
        aliases = {3: 0}
    return pl.pallas_call(
        _mm_body,
        grid=(CHUNK // M_BLK,),
        in_specs=in_specs,
        out_specs=pl.BlockSpec((M_BLK, OUT_DIM),
                               lambda i, blk0=blk0: (i + blk0, 0)),
        out_shape=jax.ShapeDtypeStruct((N_TOKENS, OUT_DIM), jnp.float32),
        input_output_aliases=aliases,
    )(*args)


@jax.jit
def kernel(ids, codebook, W_proj, b_proj, W_out, b_out, W_dec, b_dec):
    B, T = ids.shape
    gather = _sc_gather_fn()
    ids_v = ids.reshape(IDS_ROWS, 128)
    rpc = IDS_ROWS // NCHUNK
    if NCHUNK == 1:
        embs = [gather(codebook, ids_v)]
    else:
        embs = [gather(codebook,
                       lax.slice(ids_v, (c * rpc, 0), ((c + 1) * rpc, 128)))
                for c in range(NCHUNK)]
    wf, bf = _fuse_weights(W_proj, b_proj, W_out, b_out, W_dec, b_dec)
    out = _tc_decode_chunk(0, embs[0], wf, bf)
    for c in range(1, NCHUNK):
        out = _tc_decode_chunk(c, embs[c], wf, bf, out)
    return out.reshape(B, T, OUT_DIM)
